# baseline (speedup 1.0000x reference)
import jax
import jax.numpy as jnp
from jax import lax
from jax.experimental import pallas as pl
from jax.experimental.pallas import tpu as pltpu
from jax.experimental.pallas import tpu_sc as plsc

NC = 2
NS = 16
NW = NC * NS

B, L, W = 4096, 200, 32
TOTAL = B * L
CHUNK = 640
NCHUNKS = TOTAL // CHUNK
CPW = NCHUNKS // NW
GROUPS = CHUNK // 16
TROWS = 2056
PAD_ROW = 2048

_SPLAT_DN = lax.GatherDimensionNumbers(
    offset_dims=(), collapsed_slice_dims=(0,), start_index_map=(0,))


def _splat(vec, r):
    return lax.gather(vec, jnp.full((16, 1), r, jnp.int32), _SPLAT_DN,
                      slice_sizes=(1,),
                      mode=lax.GatherScatterMode.PROMISE_IN_BOUNDS)


def _sc_body(ins_hbm, table_hbm, out_hbm,
             table_v, in0, in1, rows0, rows1,
             insem0, insem1, outsem0, outsem1):
    wid = lax.axis_index("s") * NC + lax.axis_index("c")
    base = wid * CPW
    in_bufs = (in0, in1)
    row_bufs = (rows0, rows1)
    insems = (insem0, insem1)
    outsems = (outsem0, outsem1)

    def start_in(cid, slot):
        pltpu.async_copy(ins_hbm.at[cid], in_bufs[slot], insems[slot])

    start_in(base + 0, 0)
    start_in(base + 1, 1)
    pltpu.sync_copy(table_hbm, table_v)

    iota_lo = lax.iota(jnp.int32, 16)
    iota_hi = iota_lo + 16

    def do_chunk(c, slot):
        in_v = in_bufs[slot]
        rows_v = row_bufs[slot]
        pltpu.make_async_copy(ins_hbm.at[0], in_v, insems[slot]).wait()

        @pl.when(c >= base + 2)
        def _():
            pltpu.make_async_copy(rows_v, out_hbm.at[c], outsems[slot]).wait()

        def group(g, carry):
            sl = pl.ds(g * 16, 16)
            x = in_v[0, sl]
            m = in_v[1, sl]
            bases = jnp.where(m != 0, x, PAD_ROW) * W
            for r in range(16):
                row = g * 16 + r
                bvec = _splat(bases, r)
                lo = plsc.load_gather(table_v, [bvec + iota_lo])
                hi = plsc.load_gather(table_v, [bvec + iota_hi])
                rows_v[row, pl.ds(0, 16)] = lo
                rows_v[row, pl.ds(16, 16)] = hi
            return carry

        lax.fori_loop(0, GROUPS, group, 0)

        @pl.when(c + 2 < base + CPW)
        def _():
            start_in(c + 2, slot)

        pltpu.async_copy(rows_v, out_hbm.at[c], outsems[slot])

    def body(t, carry):
        do_chunk(base + 2 * t, 0)
        do_chunk(base + 2 * t + 1, 1)
        return carry

    lax.fori_loop(0, CPW // 2, body, 0)
    pltpu.make_async_copy(rows0, out_hbm.at[base], outsems[0]).wait()
    pltpu.make_async_copy(rows1, out_hbm.at[base], outsems[1]).wait()


@jax.jit
def _run(ins3, table_flat):
    mesh = plsc.VectorSubcoreMesh(core_axis_name="c", subcore_axis_name="s")
    kfn = pl.kernel(
        _sc_body,
        out_type=jax.ShapeDtypeStruct((NCHUNKS, CHUNK, W), jnp.float32),
        mesh=mesh,
        scratch_types=[
            pltpu.VMEM((TROWS * W,), jnp.float32),
            pltpu.VMEM((2, CHUNK), jnp.int32),
            pltpu.VMEM((2, CHUNK), jnp.int32),
            pltpu.VMEM((CHUNK, W), jnp.float32),
            pltpu.VMEM((CHUNK, W), jnp.float32),
            pltpu.SemaphoreType.DMA,
            pltpu.SemaphoreType.DMA,
            pltpu.SemaphoreType.DMA,
            pltpu.SemaphoreType.DMA,
        ],
        compiler_params=pltpu.CompilerParams(use_tc_tiling_on_sc=False,
                                             needs_layout_passes=False),
    )
    return kfn(ins3, table_flat)


def kernel(obs_pos, obs_mask, embedding_table):
    idx3 = obs_pos.astype(jnp.int32).reshape(NCHUNKS, CHUNK)
    mask3 = obs_mask.astype(jnp.int32).reshape(NCHUNKS, CHUNK)
    ins3 = jnp.stack([idx3, mask3], axis=1)
    table_flat = jnp.concatenate(
        [embedding_table, jnp.zeros((TROWS - 2048, W), jnp.float32)],
        axis=0).reshape(-1)
    out = _run(ins3, table_flat)
    return out.reshape(B, L, W)


# speedup vs baseline: 1.0503x; 1.0503x over previous
import jax
import jax.numpy as jnp
from jax import lax
from jax.experimental import pallas as pl
from jax.experimental.pallas import tpu as pltpu
from jax.experimental.pallas import tpu_sc as plsc

NC = 2
NS = 16
NW = NC * NS

B, L, W = 4096, 200, 32
TOTAL = B * L
SUB = 128
NSUB = 10
CHUNK = SUB * NSUB
NCHUNKS = TOTAL // CHUNK
CPW = NCHUNKS // NW
TROWS = 2056
PAD_ROW = 2048


def _sc_body(idx_hbm, mask_hbm, table_hbm, out_hbm,
             table_v, idx0, idx1, msk0, msk1, idxm0, idxm1, rows0, rows1,
             insem0, insem1, gsem0, gsem1, outsem0, outsem1):
    wid = lax.axis_index("s") * NC + lax.axis_index("c")
    base = wid * CPW
    idx_bufs = (idx0, idx1)
    msk_bufs = (msk0, msk1)
    idxm_bufs = (idxm0, idxm1)
    row_bufs = (rows0, rows1)
    insems = (insem0, insem1)
    gsems = (gsem0, gsem1)
    outsems = (outsem0, outsem1)

    def start_in(cid, slot):
        pltpu.async_copy(idx_hbm.at[cid], idx_bufs[slot], insems[slot])
        pltpu.async_copy(mask_hbm.at[cid], msk_bufs[slot], insems[slot])

    def wait_in(slot):
        pltpu.make_async_copy(idx_hbm.at[0], idx_bufs[slot],
                              insems[slot]).wait()
        pltpu.make_async_copy(mask_hbm.at[0], msk_bufs[slot],
                              insems[slot]).wait()

    start_in(base + 0, 0)
    start_in(base + 1, 1)

    @pl.when(lax.axis_index("s") == 0)
    def _():
        pltpu.sync_copy(table_hbm, table_v)

    plsc.subcore_barrier()

    def do_chunk(c, slot):
        idx_v = idx_bufs[slot]
        msk_v = msk_bufs[slot]
        idxm_v = idxm_bufs[slot]
        rows_v = row_bufs[slot]
        wait_in(slot)
        for j in range(NSUB):
            for i in range(SUB // 16):
                sl = pl.ds(i * 16, 16)
                m = msk_v[j, sl]
                x = idx_v[j, sl]
                idxm_v[j, sl] = jnp.where(m != 0, x, PAD_ROW)

        @pl.when(c + 2 < base + CPW)
        def _():
            start_in(c + 2, slot)

        @pl.when(c >= base + 2)
        def _():
            pltpu.make_async_copy(rows_v, out_hbm.at[c], outsems[slot]).wait()

        for j in range(NSUB):
            pltpu.async_copy(table_v.at[idxm_v.at[j]], rows_v.at[j],
                             gsems[slot])
        pltpu.make_async_copy(out_hbm.at[c], rows_v, gsems[slot]).wait()
        pltpu.async_copy(rows_v, out_hbm.at[c], outsems[slot])

    def body(t, carry):
        do_chunk(base + 2 * t, 0)
        do_chunk(base + 2 * t + 1, 1)
        return carry

    lax.fori_loop(0, CPW // 2, body, 0)
    pltpu.make_async_copy(rows0, out_hbm.at[base], outsems[0]).wait()
    pltpu.make_async_copy(rows1, out_hbm.at[base], outsems[1]).wait()


@jax.jit
def _run(idx2, mask2, table_pad):
    mesh = plsc.VectorSubcoreMesh(core_axis_name="c", subcore_axis_name="s")
    kfn = pl.kernel(
        _sc_body,
        out_type=jax.ShapeDtypeStruct((NCHUNKS, NSUB, SUB, W), jnp.float32),
        mesh=mesh,
        scratch_types=[
            pltpu.VMEM_SHARED((TROWS, W), jnp.float32),
            pltpu.VMEM((NSUB, SUB), jnp.int32),
            pltpu.VMEM((NSUB, SUB), jnp.int32),
            pltpu.VMEM((NSUB, SUB), jnp.int32),
            pltpu.VMEM((NSUB, SUB), jnp.int32),
            pltpu.VMEM((NSUB, SUB), jnp.int32),
            pltpu.VMEM((NSUB, SUB), jnp.int32),
            pltpu.VMEM((NSUB, SUB, W), jnp.float32),
            pltpu.VMEM((NSUB, SUB, W), jnp.float32),
            pltpu.SemaphoreType.DMA,
            pltpu.SemaphoreType.DMA,
            pltpu.SemaphoreType.DMA,
            pltpu.SemaphoreType.DMA,
            pltpu.SemaphoreType.DMA,
            pltpu.SemaphoreType.DMA,
        ],
        compiler_params=pltpu.CompilerParams(use_tc_tiling_on_sc=False),
    )
    return kfn(idx2, mask2, table_pad)


def kernel(obs_pos, obs_mask, embedding_table):
    idx2 = obs_pos.astype(jnp.int32).reshape(NCHUNKS, NSUB, SUB)
    mask2 = obs_mask.astype(jnp.int32).reshape(NCHUNKS, NSUB, SUB)
    table_pad = jnp.concatenate(
        [embedding_table, jnp.zeros((TROWS - 2048, W), jnp.float32)], axis=0)
    out = _run(idx2, mask2, table_pad)
    return out.reshape(B, L, W)
